# Initial kernel scaffold; baseline (speedup 1.0000x reference)
import jax
import jax.numpy as jnp
from jax.experimental import pallas as pl


def kernel(x, edge_index, batch, W0a, b0a, W0b, b0b, W1a, b1a, W1b, b1b, W2a, b2a, W2b, b2b):
    raise NotImplementedError("write your pallas kernel here")


# speedup vs baseline: 6.0303x; 6.0303x over previous
import functools

import jax
import jax.numpy as jnp
from jax import lax
from jax.experimental import pallas as pl
from jax.experimental.pallas import tpu as pltpu
from jax.experimental.pallas import tpu_sc as plsc

N_NODES = 10000
N_EDGES = 320000
N_GRAPHS = 128
D_IN = 128
D_EMB = 256

_NS = 16
_CH = 128
_NB = 3
_ZROWS = 640
_ACC_ROWS = _NS * _ZROWS
_TRASH = N_NODES


def _seg_sum_sc(D2, colsplit):
    mesh = plsc.VectorSubcoreMesh(core_axis_name="c", subcore_axis_name="s")
    ept = (N_EDGES if colsplit else N_EDGES // 2) // _NS
    nfull = ept // _CH
    tail = ept - nfull * _CH
    assert nfull % 2 == 0

    @functools.partial(
        pl.kernel,
        out_type=jax.ShapeDtypeStruct((2, _ACC_ROWS, D2), jnp.float32),
        mesh=mesh,
        scratch_types=[
            [pltpu.VMEM((_CH,), jnp.int32) for _ in range(2)],
            [pltpu.VMEM((_CH,), jnp.int32) for _ in range(2)],
            [pltpu.VMEM((_CH, D2), jnp.float32) for _ in range(2)],
            pltpu.VMEM_SHARED((_ACC_ROWS, D2), jnp.float32),
            pltpu.SemaphoreType.DMA,
            pltpu.SemaphoreType.DMA,
        ],
    )
    def k(src_hbm, dst_hbm, table_hbm, out_hbm,
          srcb, dstb, rows, acc, sem_l, sem_g):
        c = lax.axis_index("c")
        s = lax.axis_index("s")
        if colsplit:
            ebase = s * ept
        else:
            ebase = c * (N_EDGES // 2) + s * ept

        zero16 = jnp.zeros((16,), jnp.float32)

        def zrow(r, carry):
            for g in range(D2 // 16):
                rows[0][r, pl.ds(g * 16, 16)] = zero16
            return carry

        lax.fori_loop(0, _CH, zrow, 0)
        zb = s * _ZROWS
        for kk in range(_ZROWS // _CH):
            pltpu.sync_copy(rows[0], acc.at[pl.ds(zb + kk * _CH, _CH)])
        plsc.subcore_barrier()

        def lissue(j, b):
            base = ebase + j * _CH
            pltpu.async_copy(src_hbm.at[pl.ds(base, _CH)], srcb[b], sem_l)
            pltpu.async_copy(dst_hbm.at[pl.ds(base, _CH)], dstb[b], sem_l)

        def lwait(j, b):
            base = ebase + j * _CH
            pltpu.make_async_copy(
                src_hbm.at[pl.ds(base, _CH)], srcb[b], sem_l).wait()
            pltpu.make_async_copy(
                dst_hbm.at[pl.ds(base, _CH)], dstb[b], sem_l).wait()

        def to_idx(b):
            if colsplit:
                for g in range(_CH // 16):
                    sl = pl.ds(g * 16, 16)
                    srcb[b][sl] = srcb[b][sl] * 2 + c

        def gissue(b):
            pltpu.async_copy(table_hbm.at[srcb[b]], rows[b], sem_g)

        def gwait(b):
            pltpu.make_async_copy(
                table_hbm.at[srcb[b]], rows[b], sem_g).wait()

        def scatter(b):
            pltpu.sync_copy(rows[b], acc.at[dstb[b]], add=True)

        lissue(0, 0)
        lwait(0, 0)
        to_idx(0)
        gissue(0)
        lissue(1, 1)

        def pair(jp, carry):
            j = 2 * jp
            lwait(j + 1, 1)
            to_idx(1)
            gwait(0)
            gissue(1)
            scatter(0)

            @pl.when(j + 2 < nfull)
            def _():
                lissue(j + 2, 0)
                lwait(j + 2, 0)
                to_idx(0)

            gwait(1)

            @pl.when(j + 2 < nfull)
            def _():
                gissue(0)

            scatter(1)

            @pl.when(j + 3 < nfull)
            def _():
                lissue(j + 3, 1)

            return carry

        lax.fori_loop(0, nfull // 2, pair, 0)

        if tail:
            for g in range(_CH // 16):
                sl = pl.ds(g * 16, 16)
                srcb[0][sl] = jnp.zeros((16,), jnp.int32)
                dstb[0][sl] = jnp.full((16,), _TRASH, jnp.int32)
            tb = ebase + nfull * _CH
            pltpu.sync_copy(src_hbm.at[pl.ds(tb, tail)],
                            srcb[0].at[pl.ds(0, tail)])
            pltpu.sync_copy(dst_hbm.at[pl.ds(tb, tail)],
                            dstb[0].at[pl.ds(0, tail)])
            to_idx(0)
            pltpu.async_copy(table_hbm.at[srcb[0]], rows[0], sem_g).wait()
            scatter(0)

        plsc.subcore_barrier()

        for kk in range(_ZROWS // _CH):
            pltpu.sync_copy(acc.at[pl.ds(zb + kk * _CH, _CH)],
                            out_hbm.at[c, pl.ds(zb + kk * _CH, _CH)])

    return k


_BN = 1000


def _mlp_body(colsplit, h_ref, a_ref, wa_ref, ba_ref, wb_ref, bb_ref, o_ref):
    if colsplit:
        agg = jnp.concatenate([a_ref[0], a_ref[1]], axis=1)
    else:
        agg = a_ref[0] + a_ref[1]
    z = h_ref[...] + agg
    t = jnp.maximum(
        jnp.dot(z, wa_ref[...], preferred_element_type=jnp.float32)
        + ba_ref[...], 0.0)
    o = (jnp.dot(t, wb_ref[...], preferred_element_type=jnp.float32)
         + bb_ref[...])
    o_ref[...] = jnp.maximum(o, 0.0)


def _mlp_tc(h, agg, Wa, ba, Wb, bb, colsplit):
    Din = h.shape[1]
    D2 = agg.shape[2]
    grid = (N_NODES // _BN,)
    return pl.pallas_call(
        functools.partial(_mlp_body, colsplit),
        grid=grid,
        in_specs=[
            pl.BlockSpec((_BN, Din), lambda i: (i, 0)),
            pl.BlockSpec((2, _BN, D2), lambda i: (0, i, 0)),
            pl.BlockSpec((Din, D_EMB), lambda i: (0, 0)),
            pl.BlockSpec((1, D_EMB), lambda i: (0, 0)),
            pl.BlockSpec((D_EMB, D_EMB), lambda i: (0, 0)),
            pl.BlockSpec((1, D_EMB), lambda i: (0, 0)),
        ],
        out_specs=pl.BlockSpec((_BN, D_EMB), lambda i: (i, 0)),
        out_shape=jax.ShapeDtypeStruct((N_NODES, D_EMB), jnp.float32),
    )(h, agg, Wa, ba.reshape(1, -1), Wb, bb.reshape(1, -1))


def _pool_body(ne_ref, batch_ref, o_ref, acc, cnt):
    i = pl.program_id(0)
    oh = (lax.broadcasted_iota(jnp.int32, (N_GRAPHS, _BN), 0)
          == batch_ref[0]).astype(jnp.float32)
    part = jnp.dot(oh, ne_ref[...], preferred_element_type=jnp.float32)
    pcnt = jnp.sum(oh, axis=1, keepdims=True)

    @pl.when(i == 0)
    def _():
        acc[...] = part
        cnt[...] = pcnt

    @pl.when(i > 0)
    def _():
        acc[...] += part
        cnt[...] += pcnt

    @pl.when(i == pl.num_programs(0) - 1)
    def _():
        o_ref[...] = acc[...] / jnp.maximum(cnt[...], 1.0)


def _pool_tc(h, batch3d):
    D = h.shape[1]
    return pl.pallas_call(
        _pool_body,
        grid=(N_NODES // _BN,),
        in_specs=[
            pl.BlockSpec((_BN, D), lambda i: (i, 0)),
            pl.BlockSpec((1, 1, _BN), lambda i: (i, 0, 0)),
        ],
        out_specs=pl.BlockSpec((N_GRAPHS, D), lambda i: (0, 0)),
        out_shape=jax.ShapeDtypeStruct((N_GRAPHS, D), jnp.float32),
        scratch_shapes=[
            pltpu.VMEM((N_GRAPHS, D), jnp.float32),
            pltpu.VMEM((N_GRAPHS, 1), jnp.float32),
        ],
    )(h, batch3d)


def kernel(x, edge_index, batch,
           W0a, b0a, W0b, b0b, W1a, b1a, W1b, b1b, W2a, b2a, W2b, b2b):
    params = [(W0a, b0a, W0b, b0b), (W1a, b1a, W1b, b1b),
              (W2a, b2a, W2b, b2b)]
    h = x
    hs = []
    pooled = []
    src = edge_index[0]
    dst = edge_index[1]
    batch3d = batch.reshape(N_NODES // _BN, 1, _BN)
    first = True
    for Wa, ba, Wb, bb in params:
        if first:
            agg = _seg_sum_sc(h.shape[1], False)(src, dst, h)
        else:
            D2 = h.shape[1] // 2
            table = h.reshape(2 * N_NODES, D2)
            agg = _seg_sum_sc(D2, True)(src, dst, table)
        h = _mlp_tc(h, agg, Wa, ba, Wb, bb, colsplit=not first)
        first = False
        hs.append(h)
        pooled.append(_pool_tc(h, batch3d))
    node_embed = jnp.concatenate(hs, axis=1)
    graph_embed = jnp.concatenate(pooled, axis=1)
    return graph_embed, node_embed
